# Initial kernel scaffold; baseline (speedup 1.0000x reference)
import jax
import jax.numpy as jnp
from jax.experimental import pallas as pl


def kernel(x, edge_index, W1l, b1, W1r, W2l, b2, W2r, Wh, bh):
    raise NotImplementedError("write your pallas kernel here")


# speedup vs baseline: 2.9409x; 2.9409x over previous
import functools

import jax
import jax.numpy as jnp
from jax import lax
from jax.experimental import pallas as pl
from jax.experimental.pallas import tpu as pltpu
from jax.experimental.pallas import tpu_sc as plsc

_N = 10000
_E = 320000
_D = 128
_H = 64
_H2 = 32
_WA = 128
_NC = 2
_NS = 16
_NW = _NC * _NS
_EPT = 10240
_EPAD = _NW * _EPT
_CH = 128
_NCHUNK = _EPT // _CH
_NPAD = 10240
_ZPT = _NPAD // _NS
_BM = 1000


def _sc_segsum(width):
    mesh = plsc.VectorSubcoreMesh(core_axis_name="c", subcore_axis_name="s")

    @functools.partial(
        pl.kernel,
        mesh=mesh,
        out_type=jax.ShapeDtypeStruct((_NC, _NPAD, width), jnp.float32),
        scratch_types=[
            pltpu.VMEM((_NCHUNK, _CH), jnp.int32),
            pltpu.VMEM((_NCHUNK, _CH), jnp.int32),
            pltpu.VMEM((_CH, width), jnp.float32),
            pltpu.VMEM_SHARED((_NPAD, width), jnp.float32),
            pltpu.SemaphoreType.DMA,
        ],
    )
    def k(feat_hbm, src_hbm, dst_hbm, zeros_hbm, out_hbm,
          src_v, dst_v, rows_v, acc_sh, sem):
        c = lax.axis_index("c")
        s = lax.axis_index("s")
        wid = s * _NC + c
        pltpu.sync_copy(zeros_hbm.at[pl.ds(s * _ZPT, _ZPT)],
                        acc_sh.at[pl.ds(s * _ZPT, _ZPT)])
        pltpu.sync_copy(src_hbm.at[wid], src_v)
        pltpu.sync_copy(dst_hbm.at[wid], dst_v)
        plsc.subcore_barrier()

        def body(kc, carry):
            pltpu.async_copy(feat_hbm.at[src_v.at[kc]], rows_v, sem).wait()
            pltpu.sync_copy(rows_v, acc_sh.at[dst_v.at[kc]], add=True)
            return carry

        lax.fori_loop(0, _NCHUNK, body, 0)
        plsc.subcore_barrier()
        pltpu.sync_copy(acc_sh.at[pl.ds(s * _ZPT, _ZPT)],
                        out_hbm.at[c, pl.ds(s * _ZPT, _ZPT)])

    return k


def _tc1_body(x_ref, wl_ref, wr_ref, b_ref, aug_ref, r1_ref):
    xb = x_ref[...]
    p = jnp.dot(xb, wl_ref[...], preferred_element_type=jnp.float32)
    ones = jnp.ones((_BM, 1), jnp.float32)
    zpad = jnp.zeros((_BM, _WA - _H - 1), jnp.float32)
    aug_ref[...] = jnp.concatenate([p, ones, zpad], axis=1)
    r1_ref[...] = (jnp.dot(xb, wr_ref[...], preferred_element_type=jnp.float32)
                   + b_ref[...])


def _tc2_body(aa_ref, ab_ref, r1_ref, wl_ref, wr_ref, b_ref,
              p2_ref, r2_ref, inv_ref):
    agg = aa_ref[...] + ab_ref[...]
    inv = 1.0 / jnp.maximum(agg[:, _H:_H + 1], 1.0)
    h = jnp.maximum(agg[:, :_H] * inv + r1_ref[...], 0.0)
    p2 = jnp.dot(h, wl_ref[...], preferred_element_type=jnp.float32)
    p2_ref[...] = jnp.concatenate(
        [p2, jnp.zeros((_BM, _WA - _H2), jnp.float32)], axis=1)
    r2_ref[...] = (jnp.dot(h, wr_ref[...], preferred_element_type=jnp.float32)
                   + b_ref[...])
    inv_ref[...] = inv


def _tc3_body(aa_ref, ab_ref, inv_ref, r2_ref, wh_ref, bh_ref, out_ref):
    agg = aa_ref[:, :_H2] + ab_ref[:, :_H2]
    h2 = jnp.maximum(agg * inv_ref[...] + r2_ref[...], 0.0)
    out_ref[...] = (jnp.dot(h2, wh_ref[...], preferred_element_type=jnp.float32)
                    + bh_ref[...])


def _rows(i):
    return (i, 0)


def _rep(i):
    return (0, 0)


def kernel(x, edge_index, W1l, b1, W1r, W2l, b2, W2r, Wh, bh):
    grid = (_N // _BM,)
    pad = _EPAD - _E
    src = jnp.concatenate(
        [edge_index[0], jnp.zeros((pad,), jnp.int32)]).reshape(_NW, _NCHUNK, _CH)
    dst = jnp.concatenate(
        [edge_index[1],
         _N + (jnp.arange(pad, dtype=jnp.int32) % (_NPAD - _N))]
    ).reshape(_NW, _NCHUNK, _CH)
    zeros_a = jnp.zeros((_NPAD, _WA), jnp.float32)

    aug, r1 = pl.pallas_call(
        _tc1_body,
        grid=grid,
        in_specs=[
            pl.BlockSpec((_BM, _D), _rows),
            pl.BlockSpec((_D, _H), _rep),
            pl.BlockSpec((_D, _H), _rep),
            pl.BlockSpec((1, _H), _rep),
        ],
        out_specs=[
            pl.BlockSpec((_BM, _WA), _rows),
            pl.BlockSpec((_BM, _H), _rows),
        ],
        out_shape=[
            jax.ShapeDtypeStruct((_N, _WA), jnp.float32),
            jax.ShapeDtypeStruct((_N, _H), jnp.float32),
        ],
    )(x, W1l, W1r, b1.reshape(1, _H))

    agg1 = _sc_segsum(_WA)(aug, src, dst, zeros_a)

    p2, r2, inv = pl.pallas_call(
        _tc2_body,
        grid=grid,
        in_specs=[
            pl.BlockSpec((_BM, _WA), _rows),
            pl.BlockSpec((_BM, _WA), _rows),
            pl.BlockSpec((_BM, _H), _rows),
            pl.BlockSpec((_H, _H2), _rep),
            pl.BlockSpec((_H, _H2), _rep),
            pl.BlockSpec((1, _H2), _rep),
        ],
        out_specs=[
            pl.BlockSpec((_BM, _WA), _rows),
            pl.BlockSpec((_BM, _H2), _rows),
            pl.BlockSpec((_BM, 1), _rows),
        ],
        out_shape=[
            jax.ShapeDtypeStruct((_N, _WA), jnp.float32),
            jax.ShapeDtypeStruct((_N, _H2), jnp.float32),
            jax.ShapeDtypeStruct((_N, 1), jnp.float32),
        ],
    )(agg1[0], agg1[1], r1, W2l, W2r, b2.reshape(1, _H2))

    agg2 = _sc_segsum(_WA)(p2, src, dst, zeros_a)

    out = pl.pallas_call(
        _tc3_body,
        grid=grid,
        in_specs=[
            pl.BlockSpec((_BM, _WA), _rows),
            pl.BlockSpec((_BM, _WA), _rows),
            pl.BlockSpec((_BM, 1), _rows),
            pl.BlockSpec((_BM, _H2), _rows),
            pl.BlockSpec((_H2, 1), _rep),
            pl.BlockSpec((1, 1), _rep),
        ],
        out_specs=pl.BlockSpec((_BM, 1), _rows),
        out_shape=jax.ShapeDtypeStruct((_N, 1), jnp.float32),
    )(agg2[0], agg2[1], inv, r2, Wh, bh.reshape(1, 1))

    return out[:, 0]


# speedup vs baseline: 6.1357x; 2.0864x over previous
import functools

import jax
import jax.numpy as jnp
from jax import lax
from jax.experimental import pallas as pl
from jax.experimental.pallas import tpu as pltpu
from jax.experimental.pallas import tpu_sc as plsc

_N = 10000
_E = 320000
_D = 128
_H = 64
_H2 = 32
_WA = 128
_NC = 2
_NS = 16
_CH = 128
_RCH = _E // _CH
_CPW = (40, 120)
_C1BASE = _NS * _CPW[0]
_CPWMAX = max(_CPW)
_CHPAD = _C1BASE + _NS * _CPW[1] + _CPWMAX
_NPAD = 10112
_ZPT = _NPAD // _NS
_BM = 1000


def _sc_segsum(width, name):
    mesh = plsc.VectorSubcoreMesh(core_axis_name="c", subcore_axis_name="s")

    @functools.partial(
        pl.kernel,
        mesh=mesh,
        name=name,
        out_type=jax.ShapeDtypeStruct((_NC, _NPAD, width), jnp.float32),
        scratch_types=[
            pltpu.VMEM((_CPWMAX, _CH), jnp.int32),
            pltpu.VMEM((_CPWMAX, _CH), jnp.int32),
            pltpu.VMEM((_CH, width), jnp.float32),
            pltpu.VMEM_SHARED((_NPAD, width), jnp.float32),
            pltpu.SemaphoreType.DMA,
        ],
    )
    def k(feat_hbm, src_hbm, dst_hbm, zeros_hbm, out_hbm,
          src_v, dst_v, rows_v, acc_sh, sem):
        c = lax.axis_index("c")
        s = lax.axis_index("s")
        base = jnp.where(c == 0, s * _CPW[0], _C1BASE + s * _CPW[1])
        quota = jnp.where(c == 0, _CPW[0], _CPW[1])
        nch = jnp.clip(_RCH - base, 0, quota)
        pltpu.sync_copy(zeros_hbm.at[pl.ds(s * _ZPT, _ZPT)],
                        acc_sh.at[pl.ds(s * _ZPT, _ZPT)])
        pltpu.sync_copy(src_hbm.at[pl.ds(base, _CPWMAX)], src_v)
        pltpu.sync_copy(dst_hbm.at[pl.ds(base, _CPWMAX)], dst_v)
        plsc.subcore_barrier()

        def body(kc, carry):
            pltpu.async_copy(feat_hbm.at[src_v.at[kc]], rows_v, sem).wait()
            pltpu.sync_copy(rows_v, acc_sh.at[dst_v.at[kc]], add=True)
            return carry

        lax.fori_loop(0, nch, body, 0)
        plsc.subcore_barrier()
        pltpu.sync_copy(acc_sh.at[pl.ds(s * _ZPT, _ZPT)],
                        out_hbm.at[c, pl.ds(s * _ZPT, _ZPT)])

    return k


def _tc1_body(x_ref, wl_ref, wr_ref, b_ref, aug_ref, r1_ref):
    xb = x_ref[...]
    p = jnp.dot(xb, wl_ref[...], preferred_element_type=jnp.float32)
    ones = jnp.ones((_BM, 1), jnp.float32)
    zpad = jnp.zeros((_BM, _WA - _H - 1), jnp.float32)
    aug_ref[...] = jnp.concatenate([p, ones, zpad], axis=1)
    r1_ref[...] = (jnp.dot(xb, wr_ref[...], preferred_element_type=jnp.float32)
                   + b_ref[...])


def _tc2_body(aa_ref, ab_ref, r1_ref, wl_ref, wr_ref, b_ref,
              p2_ref, r2_ref, inv_ref):
    agg = aa_ref[...] + ab_ref[...]
    inv = 1.0 / jnp.maximum(agg[:, _H:_H + 1], 1.0)
    h = jnp.maximum(agg[:, :_H] * inv + r1_ref[...], 0.0)
    p2 = jnp.dot(h, wl_ref[...], preferred_element_type=jnp.float32)
    p2_ref[...] = jnp.concatenate(
        [p2, jnp.zeros((_BM, _WA - _H2), jnp.float32)], axis=1)
    r2_ref[...] = (jnp.dot(h, wr_ref[...], preferred_element_type=jnp.float32)
                   + b_ref[...])
    inv_ref[...] = inv


def _tc3_body(aa_ref, ab_ref, inv_ref, r2_ref, wh_ref, bh_ref, out_ref):
    agg = aa_ref[:, :_H2] + ab_ref[:, :_H2]
    h2 = jnp.maximum(agg * inv_ref[...] + r2_ref[...], 0.0)
    out_ref[...] = (jnp.dot(h2, wh_ref[...], preferred_element_type=jnp.float32)
                    + bh_ref[...])


def _rows(i):
    return (i, 0)


def _rep(i):
    return (0, 0)


def kernel(x, edge_index, W1l, b1, W1r, W2l, b2, W2r, Wh, bh):
    grid = (_N // _BM,)
    padch = jnp.zeros((_CHPAD - _RCH, _CH), jnp.int32)
    src = jnp.concatenate([edge_index[0].reshape(_RCH, _CH), padch])
    dst = jnp.concatenate([edge_index[1].reshape(_RCH, _CH), padch])
    zeros_a = jnp.zeros((_NPAD, _WA), jnp.float32)

    aug, r1 = pl.pallas_call(
        _tc1_body,
        grid=grid,
        in_specs=[
            pl.BlockSpec((_BM, _D), _rows),
            pl.BlockSpec((_D, _H), _rep),
            pl.BlockSpec((_D, _H), _rep),
            pl.BlockSpec((1, _H), _rep),
        ],
        out_specs=[
            pl.BlockSpec((_BM, _WA), _rows),
            pl.BlockSpec((_BM, _H), _rows),
        ],
        out_shape=[
            jax.ShapeDtypeStruct((_N, _WA), jnp.float32),
            jax.ShapeDtypeStruct((_N, _H), jnp.float32),
        ],
    )(x, W1l, W1r, b1.reshape(1, _H))

    agg1 = _sc_segsum(_WA, "sc_agg1")(aug, src, dst, zeros_a)

    p2, r2, inv = pl.pallas_call(
        _tc2_body,
        grid=grid,
        in_specs=[
            pl.BlockSpec((_BM, _WA), _rows),
            pl.BlockSpec((_BM, _WA), _rows),
            pl.BlockSpec((_BM, _H), _rows),
            pl.BlockSpec((_H, _H2), _rep),
            pl.BlockSpec((_H, _H2), _rep),
            pl.BlockSpec((1, _H2), _rep),
        ],
        out_specs=[
            pl.BlockSpec((_BM, _WA), _rows),
            pl.BlockSpec((_BM, _H2), _rows),
            pl.BlockSpec((_BM, 1), _rows),
        ],
        out_shape=[
            jax.ShapeDtypeStruct((_N, _WA), jnp.float32),
            jax.ShapeDtypeStruct((_N, _H2), jnp.float32),
            jax.ShapeDtypeStruct((_N, 1), jnp.float32),
        ],
    )(agg1[0], agg1[1], r1, W2l, W2r, b2.reshape(1, _H2))

    agg2 = _sc_segsum(_WA, "sc_agg2")(p2, src, dst, zeros_a)

    out = pl.pallas_call(
        _tc3_body,
        grid=grid,
        in_specs=[
            pl.BlockSpec((_BM, _WA), _rows),
            pl.BlockSpec((_BM, _WA), _rows),
            pl.BlockSpec((_BM, 1), _rows),
            pl.BlockSpec((_BM, _H2), _rows),
            pl.BlockSpec((_H2, 1), _rep),
            pl.BlockSpec((1, 1), _rep),
        ],
        out_specs=pl.BlockSpec((_BM, 1), _rows),
        out_shape=jax.ShapeDtypeStruct((_N, 1), jnp.float32),
    )(agg2[0], agg2[1], inv, r2, Wh, bh.reshape(1, 1))

    return out[:, 0]
